# baseline (speedup 1.0000x reference)
import functools

import jax
import jax.numpy as jnp
from jax import lax
from jax.experimental import pallas as pl
from jax.experimental.pallas import tpu as pltpu
from jax.experimental.pallas import tpu_sc as plsc

B, S, D = 2, 2048, 1024
M, TOPK = 8, 2
H = 4
DK, DV = 512, 1024
HDK, HDV = DK // H, DV // H
GLR = 16
GNORM = 16.0
CAP = 1024
NTOK = B * S
NSLOT = B * M
CHUNK = 256
SCALE = HDK ** -0.5


def _matmul_body(x_ref, w_ref, o_ref):
    o_ref[...] = jnp.dot(
        x_ref[...], w_ref[...],
        preferred_element_type=jnp.float32).astype(o_ref.dtype)


def _matmul(x, w, bm, bn, out_dtype=jnp.float32):
    m, k = x.shape
    _, n = w.shape
    return pl.pallas_call(
        _matmul_body,
        grid=(m // bm, n // bn),
        in_specs=[
            pl.BlockSpec((bm, k), lambda i, j: (i, 0)),
            pl.BlockSpec((k, bn), lambda i, j: (0, j)),
        ],
        out_specs=pl.BlockSpec((bm, bn), lambda i, j: (i, j)),
        out_shape=jax.ShapeDtypeStruct((m, n), out_dtype),
        compiler_params=pltpu.CompilerParams(
            dimension_semantics=("parallel", "parallel")),
    )(x, w)


def _gk_body(a_ref, w_ref, o_ref):
    z = jnp.dot(a_ref[...], w_ref[...], preferred_element_type=jnp.float32)
    ls = jnp.minimum(z, 0.0) - jnp.log1p(jnp.exp(-jnp.abs(z)))
    o_ref[...] = (ls * (1.0 / GNORM)).astype(jnp.bfloat16)


def _gk(a, w2):
    return pl.pallas_call(
        _gk_body,
        grid=(NTOK // 256,),
        in_specs=[
            pl.BlockSpec((256, GLR), lambda i: (i, 0)),
            pl.BlockSpec((GLR, DK), lambda i: (0, 0)),
        ],
        out_specs=pl.BlockSpec((256, DK), lambda i: (i, 0)),
        out_shape=jax.ShapeDtypeStruct((NTOK, DK), jnp.bfloat16),
    )(a, w2)


def _router_body(l_ref, rw_ref, gi_ref, si_ref, cnt_ref):
    b = pl.program_id(0)
    lg = l_ref[0]
    iota8 = jax.lax.broadcasted_iota(jnp.int32, (S, M), 1)
    neg = jnp.float32(-1e30)

    def top1(a):
        best = a[:, 0:1]
        bidx = jnp.zeros((S, 1), jnp.int32)
        for j in range(1, M):
            col = a[:, j:j + 1]
            better = col > best
            best = jnp.where(better, col, best)
            bidx = jnp.where(better, j, bidx)
        return best, bidx

    l1, m1 = top1(lg)
    lg2 = jnp.where(iota8 == m1, neg, lg)
    l2, m2 = top1(lg2)
    rw1 = jax.nn.sigmoid(l1 - l2)
    rw2 = 1.0 - rw1

    oh1 = (iota8 == m1).astype(jnp.float32)
    oh2 = (iota8 == m2).astype(jnp.float32)
    mask = oh1 + oh2
    c = mask
    k = 1
    while k < S:
        c = c + jnp.concatenate(
            [jnp.zeros((k, M), jnp.float32), c[: S - k, :]], axis=0)
        k *= 2
    counts = c[S - 1:S, :]
    rank = c - mask

    outs_rw, outs_gi, outs_si = [], [], []
    for oh, rw in ((oh1, rw1), (oh2, rw2)):
        r = jnp.sum(rank * oh, axis=1, keepdims=True)
        cn = jnp.sum(counts * oh, axis=1, keepdims=True)
        mm = jnp.sum(iota8.astype(jnp.float32) * oh, axis=1, keepdims=True)
        pos = r + (CAP - cn)
        valid = pos >= 0.0
        flat = (b * M + mm) * CAP + pos
        outs_rw.append(jnp.where(valid, rw, 0.0))
        outs_gi.append(jnp.where(valid, flat, 0.0).astype(jnp.int32))
        outs_si.append(
            jnp.where(valid, flat, jnp.float32(NSLOT * CAP)).astype(jnp.int32))
    rw_ref[0] = jnp.concatenate(outs_rw, axis=1)
    gi_ref[0] = jnp.concatenate(outs_gi, axis=1)
    si_ref[0] = jnp.concatenate(outs_si, axis=1)
    cnt_ref[0] = counts


def _router(logits):
    return pl.pallas_call(
        _router_body,
        grid=(B,),
        in_specs=[pl.BlockSpec((1, S, M), lambda b: (b, 0, 0))],
        out_specs=(
            pl.BlockSpec((1, S, TOPK), lambda b: (b, 0, 0)),
            pl.BlockSpec((1, S, TOPK), lambda b: (b, 0, 0)),
            pl.BlockSpec((1, S, TOPK), lambda b: (b, 0, 0)),
            pl.BlockSpec((1, 1, M), lambda b: (b, 0, 0)),
        ),
        out_shape=(
            jax.ShapeDtypeStruct((B, S, TOPK), jnp.float32),
            jax.ShapeDtypeStruct((B, S, TOPK), jnp.int32),
            jax.ShapeDtypeStruct((B, S, TOPK), jnp.int32),
            jax.ShapeDtypeStruct((B, 1, M), jnp.float32),
        ),
    )(logits)


def _gla_body(q_ref, k_ref, v_ref, g_ref, thr_ref, o_ref, st_ref):
    c = pl.program_id(1)

    @pl.when(c == 0)
    def _():
        st_ref[...] = jnp.zeros_like(st_ref)

    q = q_ref[0].astype(jnp.float32)
    k = k_ref[0].astype(jnp.float32)
    v = v_ref[0].astype(jnp.float32)
    g = g_ref[0].astype(jnp.float32)
    thr = thr_ref[0, 0, 0]
    row = (jax.lax.broadcasted_iota(jnp.int32, (CHUNK, 1), 0).astype(jnp.float32)
           + jnp.float32(CHUNK) * c.astype(jnp.float32))
    mb = row >= thr
    q = jnp.where(mb, q, 0.0)
    k = jnp.where(mb, k, 0.0)
    v = jnp.where(mb, v, 0.0)
    g = jnp.where(mb, g, 0.0)

    r2 = jax.lax.broadcasted_iota(jnp.int32, (CHUNK, CHUNK), 0)
    c2 = jax.lax.broadcasted_iota(jnp.int32, (CHUNK, CHUNK), 1)
    tri = (r2 >= c2).astype(jnp.float32)
    G = jnp.dot(tri, g, preferred_element_type=jnp.float32)
    qs = q * jnp.exp(G) * SCALE
    ks = k * jnp.exp(-G)
    tot = jnp.sum(g, axis=0, keepdims=True)
    kd = k * jnp.exp(tot - G)
    st = st_ref[...]
    os, sts = [], []
    for h in range(H):
        sl = slice(h * HDK, (h + 1) * HDK)
        qh, kh, vh = qs[:, sl], ks[:, sl], v[:, h * HDV:(h + 1) * HDV]
        A = jax.lax.dot_general(qh, kh, (((1,), (1,)), ((), ())),
                                preferred_element_type=jnp.float32) * tri
        oh = (jnp.dot(A, vh, preferred_element_type=jnp.float32)
              + jax.lax.dot_general(qh, st[:, sl], (((1,), (1,)), ((), ())),
                                    preferred_element_type=jnp.float32))
        os.append(oh)
        sts.append(jax.lax.dot_general(vh, kd[:, sl], (((0,), (0,)), ((), ())),
                                       preferred_element_type=jnp.float32))
    o_ref[0] = jnp.concatenate(os, axis=1)
    st_ref[...] = st * jnp.exp(tot) + jnp.concatenate(sts, axis=1)


def _gla(qkv, gk, thr, nseq, seqlen):
    nch = seqlen // CHUNK
    return pl.pallas_call(
        _gla_body,
        grid=(nseq, nch),
        in_specs=[
            pl.BlockSpec((1, CHUNK, DK), lambda n, c: (n, c, 0)),
            pl.BlockSpec((1, CHUNK, DK), lambda n, c: (n, c, 1)),
            pl.BlockSpec((1, CHUNK, DV), lambda n, c: (n, c, 1)),
            pl.BlockSpec((1, CHUNK, DK), lambda n, c: (n, c, 0)),
            pl.BlockSpec((1, 1, 1), lambda n, c: (n, 0, 0)),
        ],
        out_specs=pl.BlockSpec((1, CHUNK, DV), lambda n, c: (n, c, 0)),
        out_shape=jax.ShapeDtypeStruct((nseq, seqlen, DV), jnp.float32),
        scratch_shapes=[pltpu.VMEM((HDV, DK), jnp.float32)],
        compiler_params=pltpu.CompilerParams(
            dimension_semantics=("parallel", "arbitrary")),
    )(qkv, qkv, qkv, gk, thr)


_NC, _NS = 2, 16
NW = _NC * _NS
TPW = NTOK // NW
CCH = 16
NCHK = TPW // CCH


def _disp_body(qkv_hbm, gk_hbm, i0_hbm, i1_hbm, qr_hbm, gr_hbm,
               bq, bg, bi0, bi1, sem):
    wid = lax.axis_index("s") * _NC + lax.axis_index("c")
    for i in range(NCHK):
        off = wid * TPW + i * CCH
        pltpu.sync_copy(qkv_hbm.at[pl.ds(off, CCH)], bq)
        pltpu.sync_copy(gk_hbm.at[pl.ds(off, CCH)], bg)
        pltpu.sync_copy(i0_hbm.at[pl.ds(off, CCH)], bi0)
        pltpu.sync_copy(i1_hbm.at[pl.ds(off, CCH)], bi1)
        cs = [pltpu.async_copy(bq, qr_hbm.at[bi0], sem),
              pltpu.async_copy(bq, qr_hbm.at[bi1], sem),
              pltpu.async_copy(bg, gr_hbm.at[bi0], sem),
              pltpu.async_copy(bg, gr_hbm.at[bi1], sem)]
        for c in cs:
            c.wait()


def _dispatch(qkv, gk, i0, i1):
    return pl.kernel(
        _disp_body,
        out_type=(
            jax.ShapeDtypeStruct(((NSLOT + 1) * CAP, DK + DV // 2), jnp.int32),
            jax.ShapeDtypeStruct(((NSLOT + 1) * CAP, DK // 2), jnp.int32),
        ),
        mesh=plsc.VectorSubcoreMesh(core_axis_name="c", subcore_axis_name="s",
                                    num_cores=_NC, num_subcores=_NS),
        scratch_types=[
            pltpu.VMEM((CCH, DK + DV // 2), jnp.int32),
            pltpu.VMEM((CCH, DK // 2), jnp.int32),
            pltpu.VMEM((CCH,), jnp.int32),
            pltpu.VMEM((CCH,), jnp.int32),
            pltpu.SemaphoreType.DMA,
        ],
    )(qkv, gk, i0, i1)


def _comb_body(orf_hbm, i0_hbm, i1_hbm, o0_hbm, o1_hbm, b0, b1, bi0, bi1, sem):
    wid = lax.axis_index("s") * _NC + lax.axis_index("c")
    for i in range(NCHK):
        off = wid * TPW + i * CCH
        pltpu.sync_copy(i0_hbm.at[pl.ds(off, CCH)], bi0)
        pltpu.sync_copy(i1_hbm.at[pl.ds(off, CCH)], bi1)
        cs = [pltpu.async_copy(orf_hbm.at[bi0], b0, sem),
              pltpu.async_copy(orf_hbm.at[bi1], b1, sem)]
        for c in cs:
            c.wait()
        pltpu.sync_copy(b0, o0_hbm.at[pl.ds(off, CCH)])
        pltpu.sync_copy(b1, o1_hbm.at[pl.ds(off, CCH)])


def _combine(orf, i0, i1):
    return pl.kernel(
        _comb_body,
        out_type=(
            jax.ShapeDtypeStruct((NTOK, DV), jnp.float32),
            jax.ShapeDtypeStruct((NTOK, DV), jnp.float32),
        ),
        mesh=plsc.VectorSubcoreMesh(core_axis_name="c", subcore_axis_name="s",
                                    num_cores=_NC, num_subcores=_NS),
        scratch_types=[
            pltpu.VMEM((CCH, DV), jnp.float32),
            pltpu.VMEM((CCH, DV), jnp.float32),
            pltpu.VMEM((CCH,), jnp.int32),
            pltpu.VMEM((CCH,), jnp.int32),
            pltpu.SemaphoreType.DMA,
        ],
    )(orf, i0, i1)


def _epi_body(g0_ref, g1_ref, osh_ref, rw_ref, g_ref, wn_ref, wo_ref, o_ref):
    rw = rw_ref[...]
    o = g0_ref[...] * rw[:, 0:1] + g1_ref[...] * rw[:, 1:2] + osh_ref[...]
    gg = g_ref[...]
    wn = wn_ref[...]
    parts = []
    for h in range(H):
        oh = o[:, h * HDV:(h + 1) * HDV]
        ms = jnp.mean(oh * oh, axis=1, keepdims=True)
        ohn = oh * jax.lax.rsqrt(ms + 1e-5) * wn
        gh = gg[:, h * HDV:(h + 1) * HDV]
        parts.append(ohn * (gh * jax.nn.sigmoid(gh)))
    ofin = jnp.concatenate(parts, axis=1)
    o_ref[...] = jnp.dot(ofin, wo_ref[...], preferred_element_type=jnp.float32)


def _epilogue(go0, go1, osh, rw, g, wn, wo):
    bt = 256
    return pl.pallas_call(
        _epi_body,
        grid=(NTOK // bt,),
        in_specs=[
            pl.BlockSpec((bt, DV), lambda i: (i, 0)),
            pl.BlockSpec((bt, DV), lambda i: (i, 0)),
            pl.BlockSpec((bt, DV), lambda i: (i, 0)),
            pl.BlockSpec((bt, TOPK), lambda i: (i, 0)),
            pl.BlockSpec((bt, DV), lambda i: (i, 0)),
            pl.BlockSpec((1, HDV), lambda i: (0, 0)),
            pl.BlockSpec((DV, D), lambda i: (0, 0)),
        ],
        out_specs=pl.BlockSpec((bt, D), lambda i: (i, 0)),
        out_shape=jax.ShapeDtypeStruct((NTOK, D), jnp.float32),
        compiler_params=pltpu.CompilerParams(
            dimension_semantics=("parallel",)),
    )(go0, go1, osh, rw, g, wn, wo)


def kernel(hidden_states, W_router, Wq, Wk, Wv, Wgk1, Wgk2, Wg, w_norm, Wo):
    x2d = hidden_states.reshape(NTOK, D)
    wqkv = jnp.concatenate([Wq, Wk, Wv], axis=1)
    waux = jnp.concatenate(
        [Wg, Wgk1, W_router,
         jnp.zeros((D, 128 - GLR - M), jnp.float32)], axis=1)

    qkv = _matmul(x2d, wqkv, 256, 512, jnp.bfloat16)
    aux = _matmul(x2d, waux, 256, 384)
    gk_all = _gk(aux[:, DV:DV + GLR], Wgk2)
    logits = aux[:, DV + GLR:DV + GLR + M].reshape(B, S, M)

    rw, gidx, sidx, cnts = _router(logits)

    si = sidx.reshape(NTOK, TOPK)
    qkv_i = jax.lax.bitcast_convert_type(
        qkv.reshape(NTOK, DK + DV // 2, 2), jnp.int32)
    gk_i = jax.lax.bitcast_convert_type(
        gk_all.reshape(NTOK, DK // 2, 2), jnp.int32)
    qkv_rf, gk_rf = _dispatch(qkv_i, gk_i, si[:, 0], si[:, 1])
    qkv_r = jax.lax.bitcast_convert_type(
        qkv_rf, jnp.bfloat16).reshape(NSLOT + 1, CAP, 2 * DK + DV)
    gk_r = jax.lax.bitcast_convert_type(
        gk_rf, jnp.bfloat16).reshape(NSLOT + 1, CAP, DK)
    thr_r = (jnp.float32(CAP) - cnts.reshape(NSLOT)).reshape(NSLOT, 1, 1)

    o_r = _gla(qkv_r, gk_r, thr_r, NSLOT, CAP)

    o_sh = _gla(qkv.reshape(B, S, 2 * DK + DV),
                gk_all.reshape(B, S, DK),
                jnp.zeros((B, 1, 1), jnp.float32), B, S).reshape(NTOK, DV)

    o_rf = o_r.reshape(NSLOT * CAP, DV)
    gi = gidx.reshape(NTOK, TOPK)
    go0, go1 = _combine(o_rf, gi[:, 0], gi[:, 1])

    out = _epilogue(go0, go1, o_sh, rw.reshape(NTOK, TOPK), aux,
                    w_norm.reshape(1, HDV), Wo)
    return out.reshape(B, S, D)


# speedup vs baseline: 2.7630x; 2.7630x over previous
import functools

import jax
import jax.numpy as jnp
from jax import lax
from jax.experimental import pallas as pl
from jax.experimental.pallas import tpu as pltpu
from jax.experimental.pallas import tpu_sc as plsc

B, S, D = 2, 2048, 1024
M, TOPK = 8, 2
H = 4
DK, DV = 512, 1024
HDK, HDV = DK // H, DV // H
GLR = 16
GNORM = 16.0
CAP = 1024
NTOK = B * S
NSLOT = B * M
CHUNK = 512
SCALE = HDK ** -0.5


def _matmul_body(x_ref, w_ref, o_ref):
    o_ref[...] = jnp.dot(x_ref[...], w_ref[...], preferred_element_type=jnp.float32)


def _matmul(x, w, bm, bn):
    m, k = x.shape
    _, n = w.shape
    return pl.pallas_call(
        _matmul_body,
        grid=(m // bm, n // bn),
        in_specs=[
            pl.BlockSpec((bm, k), lambda i, j: (i, 0)),
            pl.BlockSpec((k, bn), lambda i, j: (0, j)),
        ],
        out_specs=pl.BlockSpec((bm, bn), lambda i, j: (i, j)),
        out_shape=jax.ShapeDtypeStruct((m, n), jnp.float32),
        compiler_params=pltpu.CompilerParams(
            dimension_semantics=("parallel", "parallel")),
    )(x, w)


def _gk_body(a_ref, w_ref, o_ref):
    z = jnp.dot(a_ref[...], w_ref[...], preferred_element_type=jnp.float32)
    ls = jnp.minimum(z, 0.0) - jnp.log1p(jnp.exp(-jnp.abs(z)))
    o_ref[...] = ls * (1.0 / GNORM)


def _gk(a, w2):
    return pl.pallas_call(
        _gk_body,
        grid=(NTOK // 256,),
        in_specs=[
            pl.BlockSpec((256, GLR), lambda i: (i, 0)),
            pl.BlockSpec((GLR, DK), lambda i: (0, 0)),
        ],
        out_specs=pl.BlockSpec((256, DK), lambda i: (i, 0)),
        out_shape=jax.ShapeDtypeStruct((NTOK, DK), jnp.float32),
    )(a, w2)


def _router_body(l_ref, rw_ref, gi_ref, si_ref, cnt_ref):
    b = pl.program_id(0)
    lg = l_ref[0]
    iota8 = jax.lax.broadcasted_iota(jnp.int32, (S, M), 1)
    neg = jnp.float32(-1e30)

    def top1(a):
        best = a[:, 0:1]
        bidx = jnp.zeros((S, 1), jnp.int32)
        for j in range(1, M):
            col = a[:, j:j + 1]
            better = col > best
            best = jnp.where(better, col, best)
            bidx = jnp.where(better, j, bidx)
        return best, bidx

    l1, m1 = top1(lg)
    lg2 = jnp.where(iota8 == m1, neg, lg)
    l2, m2 = top1(lg2)
    rw1 = jax.nn.sigmoid(l1 - l2)
    rw2 = 1.0 - rw1

    oh1 = (iota8 == m1).astype(jnp.float32)
    oh2 = (iota8 == m2).astype(jnp.float32)
    mask = oh1 + oh2
    c = mask
    k = 1
    while k < S:
        c = c + jnp.concatenate(
            [jnp.zeros((k, M), jnp.float32), c[: S - k, :]], axis=0)
        k *= 2
    counts = c[S - 1:S, :]
    rank = c - mask

    outs_rw, outs_gi, outs_si = [], [], []
    for oh, rw in ((oh1, rw1), (oh2, rw2)):
        r = jnp.sum(rank * oh, axis=1, keepdims=True)
        cn = jnp.sum(counts * oh, axis=1, keepdims=True)
        mm = jnp.sum(iota8.astype(jnp.float32) * oh, axis=1, keepdims=True)
        pos = r + (CAP - cn)
        valid = pos >= 0.0
        flat = (b * M + mm) * CAP + pos
        outs_rw.append(jnp.where(valid, rw, 0.0))
        outs_gi.append(jnp.where(valid, flat, 0.0).astype(jnp.int32))
        outs_si.append(
            jnp.where(valid, flat, jnp.float32(NSLOT * CAP)).astype(jnp.int32))
    rw_ref[0] = jnp.concatenate(outs_rw, axis=1)
    gi_ref[0] = jnp.concatenate(outs_gi, axis=1)
    si_ref[0] = jnp.concatenate(outs_si, axis=1)
    cnt_ref[0] = counts


def _router(logits):
    return pl.pallas_call(
        _router_body,
        grid=(B,),
        in_specs=[pl.BlockSpec((1, S, M), lambda b: (b, 0, 0))],
        out_specs=(
            pl.BlockSpec((1, S, TOPK), lambda b: (b, 0, 0)),
            pl.BlockSpec((1, S, TOPK), lambda b: (b, 0, 0)),
            pl.BlockSpec((1, S, TOPK), lambda b: (b, 0, 0)),
            pl.BlockSpec((1, 1, M), lambda b: (b, 0, 0)),
        ),
        out_shape=(
            jax.ShapeDtypeStruct((B, S, TOPK), jnp.float32),
            jax.ShapeDtypeStruct((B, S, TOPK), jnp.int32),
            jax.ShapeDtypeStruct((B, S, TOPK), jnp.int32),
            jax.ShapeDtypeStruct((B, 1, M), jnp.float32),
        ),
    )(logits)


def _gla_body(q_ref, k_ref, v_ref, g_ref, thr_ref, o_ref, st_ref):
    c = pl.program_id(1)

    @pl.when(c == 0)
    def _():
        st_ref[...] = jnp.zeros_like(st_ref)

    q = q_ref[0]
    k = k_ref[0]
    v = v_ref[0]
    g = g_ref[0]
    thr = thr_ref[0, 0, 0]
    row = (jax.lax.broadcasted_iota(jnp.int32, (CHUNK, 1), 0).astype(jnp.float32)
           + jnp.float32(CHUNK) * c.astype(jnp.float32))
    mb = row >= thr
    q = jnp.where(mb, q, 0.0)
    k = jnp.where(mb, k, 0.0)
    v = jnp.where(mb, v, 0.0)
    g = jnp.where(mb, g, 0.0)

    r2 = jax.lax.broadcasted_iota(jnp.int32, (CHUNK, CHUNK), 0)
    c2 = jax.lax.broadcasted_iota(jnp.int32, (CHUNK, CHUNK), 1)
    tri = (r2 >= c2).astype(jnp.float32)
    G = jnp.dot(tri, g, preferred_element_type=jnp.float32)
    qs = q * jnp.exp(G) * SCALE
    ks = k * jnp.exp(-G)
    tot = jnp.sum(g, axis=0, keepdims=True)
    kd = k * jnp.exp(tot - G)
    st = st_ref[...]
    os, sts = [], []
    for h in range(H):
        sl = slice(h * HDK, (h + 1) * HDK)
        qh, kh, vh = qs[:, sl], ks[:, sl], v[:, h * HDV:(h + 1) * HDV]
        A = jax.lax.dot_general(qh, kh, (((1,), (1,)), ((), ())),
                                preferred_element_type=jnp.float32) * tri
        oh = (jnp.dot(A, vh, preferred_element_type=jnp.float32)
              + jax.lax.dot_general(qh, st[:, sl], (((1,), (1,)), ((), ())),
                                    preferred_element_type=jnp.float32))
        os.append(oh)
        sts.append(jax.lax.dot_general(vh, kd[:, sl], (((0,), (0,)), ((), ())),
                                       preferred_element_type=jnp.float32))
    o_ref[0] = jnp.concatenate(os, axis=1)
    st_ref[...] = st * jnp.exp(tot) + jnp.concatenate(sts, axis=1)


def _gla(qkv, gk, thr, nseq, seqlen):
    nch = seqlen // CHUNK
    return pl.pallas_call(
        _gla_body,
        grid=(nseq, nch),
        in_specs=[
            pl.BlockSpec((1, CHUNK, DK), lambda n, c: (n, c, 0)),
            pl.BlockSpec((1, CHUNK, DK), lambda n, c: (n, c, 1)),
            pl.BlockSpec((1, CHUNK, DV), lambda n, c: (n, c, 1)),
            pl.BlockSpec((1, CHUNK, DK), lambda n, c: (n, c, 0)),
            pl.BlockSpec((1, 1, 1), lambda n, c: (n, 0, 0)),
        ],
        out_specs=pl.BlockSpec((1, CHUNK, DV), lambda n, c: (n, c, 0)),
        out_shape=jax.ShapeDtypeStruct((nseq, seqlen, DV), jnp.float32),
        scratch_shapes=[pltpu.VMEM((HDV, DK), jnp.float32)],
        compiler_params=pltpu.CompilerParams(
            dimension_semantics=("parallel", "arbitrary")),
    )(qkv, qkv, qkv, gk, thr)


_NC, _NS = 2, 16
NW = _NC * _NS
TPW = NTOK // NW
CCH = 16
NCHK = TPW // CCH


def _disp_body(qkv_hbm, gk_hbm, i0_hbm, i1_hbm, qr_hbm, gr_hbm,
               bq, bg, bi0, bi1, sem):
    wid = lax.axis_index("s") * _NC + lax.axis_index("c")
    for i in range(NCHK):
        off = wid * TPW + i * CCH
        pltpu.sync_copy(qkv_hbm.at[pl.ds(off, CCH)], bq)
        pltpu.sync_copy(gk_hbm.at[pl.ds(off, CCH)], bg)
        pltpu.sync_copy(i0_hbm.at[pl.ds(off, CCH)], bi0)
        pltpu.sync_copy(i1_hbm.at[pl.ds(off, CCH)], bi1)
        cs = [pltpu.async_copy(bq, qr_hbm.at[bi0], sem),
              pltpu.async_copy(bq, qr_hbm.at[bi1], sem),
              pltpu.async_copy(bg, gr_hbm.at[bi0], sem),
              pltpu.async_copy(bg, gr_hbm.at[bi1], sem)]
        for c in cs:
            c.wait()


def _dispatch(qkv, gk, i0, i1):
    return pl.kernel(
        _disp_body,
        out_type=(
            jax.ShapeDtypeStruct(((NSLOT + 1) * CAP, 2 * DK + DV), jnp.float32),
            jax.ShapeDtypeStruct(((NSLOT + 1) * CAP, DK), jnp.float32),
        ),
        mesh=plsc.VectorSubcoreMesh(core_axis_name="c", subcore_axis_name="s",
                                    num_cores=_NC, num_subcores=_NS),
        scratch_types=[
            pltpu.VMEM((CCH, 2 * DK + DV), jnp.float32),
            pltpu.VMEM((CCH, DK), jnp.float32),
            pltpu.VMEM((CCH,), jnp.int32),
            pltpu.VMEM((CCH,), jnp.int32),
            pltpu.SemaphoreType.DMA,
        ],
    )(qkv, gk, i0, i1)


def _comb_body(orf_hbm, i0_hbm, i1_hbm, o0_hbm, o1_hbm, b0, b1, bi0, bi1, sem):
    wid = lax.axis_index("s") * _NC + lax.axis_index("c")
    for i in range(NCHK):
        off = wid * TPW + i * CCH
        pltpu.sync_copy(i0_hbm.at[pl.ds(off, CCH)], bi0)
        pltpu.sync_copy(i1_hbm.at[pl.ds(off, CCH)], bi1)
        cs = [pltpu.async_copy(orf_hbm.at[bi0], b0, sem),
              pltpu.async_copy(orf_hbm.at[bi1], b1, sem)]
        for c in cs:
            c.wait()
        pltpu.sync_copy(b0, o0_hbm.at[pl.ds(off, CCH)])
        pltpu.sync_copy(b1, o1_hbm.at[pl.ds(off, CCH)])


def _combine(orf, i0, i1):
    return pl.kernel(
        _comb_body,
        out_type=(
            jax.ShapeDtypeStruct((NTOK, DV), jnp.float32),
            jax.ShapeDtypeStruct((NTOK, DV), jnp.float32),
        ),
        mesh=plsc.VectorSubcoreMesh(core_axis_name="c", subcore_axis_name="s",
                                    num_cores=_NC, num_subcores=_NS),
        scratch_types=[
            pltpu.VMEM((CCH, DV), jnp.float32),
            pltpu.VMEM((CCH, DV), jnp.float32),
            pltpu.VMEM((CCH,), jnp.int32),
            pltpu.VMEM((CCH,), jnp.int32),
            pltpu.SemaphoreType.DMA,
        ],
    )(orf, i0, i1)


def _epi_body(g0_ref, g1_ref, osh_ref, rw_ref, g_ref, wn_ref, wo_ref, o_ref):
    rw = rw_ref[...]
    o = g0_ref[...] * rw[:, 0:1] + g1_ref[...] * rw[:, 1:2] + osh_ref[...]
    gg = g_ref[...]
    wn = wn_ref[...]
    parts = []
    for h in range(H):
        oh = o[:, h * HDV:(h + 1) * HDV]
        ms = jnp.mean(oh * oh, axis=1, keepdims=True)
        ohn = oh * jax.lax.rsqrt(ms + 1e-5) * wn
        gh = gg[:, h * HDV:(h + 1) * HDV]
        parts.append(ohn * (gh * jax.nn.sigmoid(gh)))
    ofin = jnp.concatenate(parts, axis=1)
    o_ref[...] = jnp.dot(ofin, wo_ref[...], preferred_element_type=jnp.float32)


def _epilogue(go0, go1, osh, rw, g, wn, wo):
    bt = 256
    return pl.pallas_call(
        _epi_body,
        grid=(NTOK // bt,),
        in_specs=[
            pl.BlockSpec((bt, DV), lambda i: (i, 0)),
            pl.BlockSpec((bt, DV), lambda i: (i, 0)),
            pl.BlockSpec((bt, DV), lambda i: (i, 0)),
            pl.BlockSpec((bt, TOPK), lambda i: (i, 0)),
            pl.BlockSpec((bt, DV), lambda i: (i, 0)),
            pl.BlockSpec((1, HDV), lambda i: (0, 0)),
            pl.BlockSpec((DV, D), lambda i: (0, 0)),
        ],
        out_specs=pl.BlockSpec((bt, D), lambda i: (i, 0)),
        out_shape=jax.ShapeDtypeStruct((NTOK, D), jnp.float32),
        compiler_params=pltpu.CompilerParams(
            dimension_semantics=("parallel",)),
    )(go0, go1, osh, rw, g, wn, wo)


def kernel(hidden_states, W_router, Wq, Wk, Wv, Wgk1, Wgk2, Wg, w_norm, Wo):
    x2d = hidden_states.reshape(NTOK, D)
    wqkv = jnp.concatenate([Wq, Wk, Wv], axis=1)
    waux = jnp.concatenate(
        [Wg, Wgk1, W_router,
         jnp.zeros((D, 128 - GLR - M), jnp.float32)], axis=1)

    qkv = _matmul(x2d, wqkv, 256, 512)
    aux = _matmul(x2d, waux, 256, 384)
    gk_all = _gk(aux[:, DV:DV + GLR], Wgk2)
    logits = aux[:, DV + GLR:DV + GLR + M].reshape(B, S, M)

    rw, gidx, sidx, cnts = _router(logits)

    si = sidx.reshape(NTOK, TOPK)
    qkv_rf, gk_rf = _dispatch(qkv, gk_all, si[:, 0], si[:, 1])
    qkv_r = qkv_rf.reshape(NSLOT + 1, CAP, 2 * DK + DV)
    gk_r = gk_rf.reshape(NSLOT + 1, CAP, DK)
    thr_r = (jnp.float32(CAP) - cnts.reshape(NSLOT)).reshape(NSLOT, 1, 1)

    o_r = _gla(qkv_r, gk_r, thr_r, NSLOT, CAP)

    o_sh = _gla(qkv.reshape(B, S, 2 * DK + DV),
                gk_all.reshape(B, S, DK),
                jnp.zeros((B, 1, 1), jnp.float32), B, S).reshape(NTOK, DV)

    o_rf = o_r.reshape(NSLOT * CAP, DV)
    gi = gidx.reshape(NTOK, TOPK)
    go0, go1 = _combine(o_rf, gi[:, 0], gi[:, 1])

    out = _epilogue(go0, go1, o_sh, rw.reshape(NTOK, TOPK), aux,
                    w_norm.reshape(1, HDV), Wo)
    return out.reshape(B, S, D)


# speedup vs baseline: 2.8446x; 1.0295x over previous
import functools

import jax
import jax.numpy as jnp
from jax import lax
from jax.experimental import pallas as pl
from jax.experimental.pallas import tpu as pltpu
from jax.experimental.pallas import tpu_sc as plsc

B, S, D = 2, 2048, 1024
M, TOPK = 8, 2
H = 4
DK, DV = 512, 1024
HDK, HDV = DK // H, DV // H
GLR = 16
GNORM = 16.0
CAP = 1024
NTOK = B * S
NSLOT = B * M
CHUNK = 512
SCALE = HDK ** -0.5


def _matmul_body(x_ref, w_ref, o_ref):
    o_ref[...] = jnp.dot(x_ref[...], w_ref[...], preferred_element_type=jnp.float32)


def _matmul(x, w, bm, bn):
    m, k = x.shape
    _, n = w.shape
    return pl.pallas_call(
        _matmul_body,
        grid=(m // bm, n // bn),
        in_specs=[
            pl.BlockSpec((bm, k), lambda i, j: (i, 0)),
            pl.BlockSpec((k, bn), lambda i, j: (0, j)),
        ],
        out_specs=pl.BlockSpec((bm, bn), lambda i, j: (i, j)),
        out_shape=jax.ShapeDtypeStruct((m, n), jnp.float32),
        compiler_params=pltpu.CompilerParams(
            dimension_semantics=("parallel", "parallel")),
    )(x, w)


def _gk_body(a_ref, w_ref, o_ref):
    z = jnp.dot(a_ref[...], w_ref[...], preferred_element_type=jnp.float32)
    ls = jnp.minimum(z, 0.0) - jnp.log1p(jnp.exp(-jnp.abs(z)))
    o_ref[...] = ls * (1.0 / GNORM)


def _gk(a, w2):
    return pl.pallas_call(
        _gk_body,
        grid=(NTOK // 256,),
        in_specs=[
            pl.BlockSpec((256, GLR), lambda i: (i, 0)),
            pl.BlockSpec((GLR, DK), lambda i: (0, 0)),
        ],
        out_specs=pl.BlockSpec((256, DK), lambda i: (i, 0)),
        out_shape=jax.ShapeDtypeStruct((NTOK, DK), jnp.float32),
    )(a, w2)


def _router_body(l_ref, rw_ref, gi_ref, si_ref, cnt_ref):
    b = pl.program_id(0)
    lg = l_ref[0]
    iota8 = jax.lax.broadcasted_iota(jnp.int32, (S, M), 1)
    neg = jnp.float32(-1e30)

    def top1(a):
        best = a[:, 0:1]
        bidx = jnp.zeros((S, 1), jnp.int32)
        for j in range(1, M):
            col = a[:, j:j + 1]
            better = col > best
            best = jnp.where(better, col, best)
            bidx = jnp.where(better, j, bidx)
        return best, bidx

    l1, m1 = top1(lg)
    lg2 = jnp.where(iota8 == m1, neg, lg)
    l2, m2 = top1(lg2)
    rw1 = jax.nn.sigmoid(l1 - l2)
    rw2 = 1.0 - rw1

    oh1 = (iota8 == m1).astype(jnp.float32)
    oh2 = (iota8 == m2).astype(jnp.float32)
    mask = oh1 + oh2
    c = mask
    k = 1
    while k < S:
        c = c + jnp.concatenate(
            [jnp.zeros((k, M), jnp.float32), c[: S - k, :]], axis=0)
        k *= 2
    counts = c[S - 1:S, :]
    rank = c - mask

    outs_rw, outs_gi, outs_si = [], [], []
    for oh, rw in ((oh1, rw1), (oh2, rw2)):
        r = jnp.sum(rank * oh, axis=1, keepdims=True)
        cn = jnp.sum(counts * oh, axis=1, keepdims=True)
        mm = jnp.sum(iota8.astype(jnp.float32) * oh, axis=1, keepdims=True)
        pos = r + (CAP - cn)
        valid = pos >= 0.0
        flat = (b * M + mm) * CAP + pos
        outs_rw.append(jnp.where(valid, rw, 0.0))
        outs_gi.append(jnp.where(valid, flat, 0.0).astype(jnp.int32))
        outs_si.append(
            jnp.where(valid, flat, jnp.float32(NSLOT * CAP)).astype(jnp.int32))
    rw_ref[0] = jnp.concatenate(outs_rw, axis=1)
    gi_ref[0] = jnp.concatenate(outs_gi, axis=1)
    si_ref[0] = jnp.concatenate(outs_si, axis=1)
    cnt_ref[0] = counts


def _router(logits):
    return pl.pallas_call(
        _router_body,
        grid=(B,),
        in_specs=[pl.BlockSpec((1, S, M), lambda b: (b, 0, 0))],
        out_specs=(
            pl.BlockSpec((1, S, TOPK), lambda b: (b, 0, 0)),
            pl.BlockSpec((1, S, TOPK), lambda b: (b, 0, 0)),
            pl.BlockSpec((1, S, TOPK), lambda b: (b, 0, 0)),
            pl.BlockSpec((1, 1, M), lambda b: (b, 0, 0)),
        ),
        out_shape=(
            jax.ShapeDtypeStruct((B, S, TOPK), jnp.float32),
            jax.ShapeDtypeStruct((B, S, TOPK), jnp.int32),
            jax.ShapeDtypeStruct((B, S, TOPK), jnp.int32),
            jax.ShapeDtypeStruct((B, 1, M), jnp.float32),
        ),
    )(logits)


def _gla_body(q_ref, k_ref, v_ref, g_ref, thr_ref, o_ref, st_ref):
    c = pl.program_id(1)

    @pl.when(c == 0)
    def _():
        st_ref[...] = jnp.zeros_like(st_ref)

    q = q_ref[0]
    k = k_ref[0]
    v = v_ref[0]
    g = g_ref[0]
    thr = thr_ref[0, 0, 0]
    row = (jax.lax.broadcasted_iota(jnp.int32, (CHUNK, 1), 0).astype(jnp.float32)
           + jnp.float32(CHUNK) * c.astype(jnp.float32))
    mb = row >= thr
    q = jnp.where(mb, q, 0.0)
    k = jnp.where(mb, k, 0.0)
    v = jnp.where(mb, v, 0.0)
    g = jnp.where(mb, g, 0.0)

    r2 = jax.lax.broadcasted_iota(jnp.int32, (CHUNK, CHUNK), 0)
    c2 = jax.lax.broadcasted_iota(jnp.int32, (CHUNK, CHUNK), 1)
    tri = (r2 >= c2).astype(jnp.float32)
    G = jnp.dot(tri, g, preferred_element_type=jnp.float32)
    qs = q * jnp.exp(G) * SCALE
    ks = k * jnp.exp(-G)
    tot = jnp.sum(g, axis=0, keepdims=True)
    kd = k * jnp.exp(tot - G)
    st = st_ref[...]
    os, sts = [], []
    for h in range(H):
        sl = slice(h * HDK, (h + 1) * HDK)
        qh, kh, vh = qs[:, sl], ks[:, sl], v[:, h * HDV:(h + 1) * HDV]
        A = jax.lax.dot_general(qh, kh, (((1,), (1,)), ((), ())),
                                preferred_element_type=jnp.float32) * tri
        oh = (jnp.dot(A, vh, preferred_element_type=jnp.float32)
              + jax.lax.dot_general(qh, st[:, sl], (((1,), (1,)), ((), ())),
                                    preferred_element_type=jnp.float32))
        os.append(oh)
        sts.append(jax.lax.dot_general(vh, kd[:, sl], (((0,), (0,)), ((), ())),
                                       preferred_element_type=jnp.float32))
    o_ref[0] = jnp.concatenate(os, axis=1)
    st_ref[...] = st * jnp.exp(tot) + jnp.concatenate(sts, axis=1)


def _gla(qkv, gk, thr, nseq, seqlen):
    nch = seqlen // CHUNK
    return pl.pallas_call(
        _gla_body,
        grid=(nseq, nch),
        in_specs=[
            pl.BlockSpec((1, CHUNK, DK), lambda n, c: (n, c, 0)),
            pl.BlockSpec((1, CHUNK, DK), lambda n, c: (n, c, 1)),
            pl.BlockSpec((1, CHUNK, DV), lambda n, c: (n, c, 1)),
            pl.BlockSpec((1, CHUNK, DK), lambda n, c: (n, c, 0)),
            pl.BlockSpec((1, 1, 1), lambda n, c: (n, 0, 0)),
        ],
        out_specs=pl.BlockSpec((1, CHUNK, DV), lambda n, c: (n, c, 0)),
        out_shape=jax.ShapeDtypeStruct((nseq, seqlen, DV), jnp.float32),
        scratch_shapes=[pltpu.VMEM((HDV, DK), jnp.float32)],
        compiler_params=pltpu.CompilerParams(
            dimension_semantics=("parallel", "arbitrary")),
    )(qkv, qkv, qkv, gk, thr)


_NC, _NS = 2, 16
NW = _NC * _NS
TPW = NTOK // NW
CCH = 32
NCHK = TPW // CCH


def _disp_body(qkv_hbm, gk_hbm, i0_hbm, i1_hbm, qr_hbm, gr_hbm,
               bq, bg, bi0, bi1, sem):
    wid = lax.axis_index("s") * _NC + lax.axis_index("c")
    for i in range(NCHK):
        off = wid * TPW + i * CCH
        pltpu.sync_copy(qkv_hbm.at[pl.ds(off, CCH)], bq)
        pltpu.sync_copy(gk_hbm.at[pl.ds(off, CCH)], bg)
        pltpu.sync_copy(i0_hbm.at[pl.ds(off, CCH)], bi0)
        pltpu.sync_copy(i1_hbm.at[pl.ds(off, CCH)], bi1)
        cs = [pltpu.async_copy(bq, qr_hbm.at[bi0], sem),
              pltpu.async_copy(bq, qr_hbm.at[bi1], sem),
              pltpu.async_copy(bg, gr_hbm.at[bi0], sem),
              pltpu.async_copy(bg, gr_hbm.at[bi1], sem)]
        for c in cs:
            c.wait()


def _dispatch(qkv, gk, i0, i1):
    return pl.kernel(
        _disp_body,
        out_type=(
            jax.ShapeDtypeStruct(((NSLOT + 1) * CAP, 2 * DK + DV), jnp.float32),
            jax.ShapeDtypeStruct(((NSLOT + 1) * CAP, DK), jnp.float32),
        ),
        mesh=plsc.VectorSubcoreMesh(core_axis_name="c", subcore_axis_name="s",
                                    num_cores=_NC, num_subcores=_NS),
        scratch_types=[
            pltpu.VMEM((CCH, 2 * DK + DV), jnp.float32),
            pltpu.VMEM((CCH, DK), jnp.float32),
            pltpu.VMEM((CCH,), jnp.int32),
            pltpu.VMEM((CCH,), jnp.int32),
            pltpu.SemaphoreType.DMA,
        ],
    )(qkv, gk, i0, i1)


def _comb_body(orf_hbm, i0_hbm, i1_hbm, o0_hbm, o1_hbm, b0, b1, bi0, bi1, sem):
    wid = lax.axis_index("s") * _NC + lax.axis_index("c")
    for i in range(NCHK):
        off = wid * TPW + i * CCH
        pltpu.sync_copy(i0_hbm.at[pl.ds(off, CCH)], bi0)
        pltpu.sync_copy(i1_hbm.at[pl.ds(off, CCH)], bi1)
        cs = [pltpu.async_copy(orf_hbm.at[bi0], b0, sem),
              pltpu.async_copy(orf_hbm.at[bi1], b1, sem)]
        for c in cs:
            c.wait()
        pltpu.sync_copy(b0, o0_hbm.at[pl.ds(off, CCH)])
        pltpu.sync_copy(b1, o1_hbm.at[pl.ds(off, CCH)])


def _combine(orf, i0, i1):
    return pl.kernel(
        _comb_body,
        out_type=(
            jax.ShapeDtypeStruct((NTOK, DV), jnp.float32),
            jax.ShapeDtypeStruct((NTOK, DV), jnp.float32),
        ),
        mesh=plsc.VectorSubcoreMesh(core_axis_name="c", subcore_axis_name="s",
                                    num_cores=_NC, num_subcores=_NS),
        scratch_types=[
            pltpu.VMEM((CCH, DV), jnp.float32),
            pltpu.VMEM((CCH, DV), jnp.float32),
            pltpu.VMEM((CCH,), jnp.int32),
            pltpu.VMEM((CCH,), jnp.int32),
            pltpu.SemaphoreType.DMA,
        ],
    )(orf, i0, i1)


def _epi_body(g0_ref, g1_ref, osh_ref, rw_ref, g_ref, wn_ref, wo_ref, o_ref):
    rw = rw_ref[...]
    o = g0_ref[...] * rw[:, 0:1] + g1_ref[...] * rw[:, 1:2] + osh_ref[...]
    gg = g_ref[...]
    wn = wn_ref[...]
    parts = []
    for h in range(H):
        oh = o[:, h * HDV:(h + 1) * HDV]
        ms = jnp.mean(oh * oh, axis=1, keepdims=True)
        ohn = oh * jax.lax.rsqrt(ms + 1e-5) * wn
        gh = gg[:, h * HDV:(h + 1) * HDV]
        parts.append(ohn * (gh * jax.nn.sigmoid(gh)))
    ofin = jnp.concatenate(parts, axis=1)
    o_ref[...] = jnp.dot(ofin, wo_ref[...], preferred_element_type=jnp.float32)


def _epilogue(go0, go1, osh, rw, g, wn, wo):
    bt = 256
    return pl.pallas_call(
        _epi_body,
        grid=(NTOK // bt,),
        in_specs=[
            pl.BlockSpec((bt, DV), lambda i: (i, 0)),
            pl.BlockSpec((bt, DV), lambda i: (i, 0)),
            pl.BlockSpec((bt, DV), lambda i: (i, 0)),
            pl.BlockSpec((bt, TOPK), lambda i: (i, 0)),
            pl.BlockSpec((bt, DV), lambda i: (i, 0)),
            pl.BlockSpec((1, HDV), lambda i: (0, 0)),
            pl.BlockSpec((DV, D), lambda i: (0, 0)),
        ],
        out_specs=pl.BlockSpec((bt, D), lambda i: (i, 0)),
        out_shape=jax.ShapeDtypeStruct((NTOK, D), jnp.float32),
        compiler_params=pltpu.CompilerParams(
            dimension_semantics=("parallel",)),
    )(go0, go1, osh, rw, g, wn, wo)


def kernel(hidden_states, W_router, Wq, Wk, Wv, Wgk1, Wgk2, Wg, w_norm, Wo):
    x2d = hidden_states.reshape(NTOK, D)
    wqkv = jnp.concatenate([Wq, Wk, Wv], axis=1)
    waux = jnp.concatenate(
        [Wg, Wgk1, W_router,
         jnp.zeros((D, 128 - GLR - M), jnp.float32)], axis=1)

    qkv = _matmul(x2d, wqkv, 256, 512)
    aux = _matmul(x2d, waux, 256, 384)
    gk_all = _gk(aux[:, DV:DV + GLR], Wgk2)
    logits = aux[:, DV + GLR:DV + GLR + M].reshape(B, S, M)

    rw, gidx, sidx, cnts = _router(logits)

    si = sidx.reshape(NTOK, TOPK)
    qkv_rf, gk_rf = _dispatch(qkv, gk_all, si[:, 0], si[:, 1])
    qkv_r = qkv_rf.reshape(NSLOT + 1, CAP, 2 * DK + DV)
    gk_r = gk_rf.reshape(NSLOT + 1, CAP, DK)
    thr_r = (jnp.float32(CAP) - cnts.reshape(NSLOT)).reshape(NSLOT, 1, 1)

    o_r = _gla(qkv_r, gk_r, thr_r, NSLOT, CAP)

    o_sh = _gla(qkv.reshape(B, S, 2 * DK + DV),
                gk_all.reshape(B, S, DK),
                jnp.zeros((B, 1, 1), jnp.float32), B, S).reshape(NTOK, DV)

    o_rf = o_r.reshape(NSLOT * CAP, DV)
    gi = gidx.reshape(NTOK, TOPK)
    go0, go1 = _combine(o_rf, gi[:, 0], gi[:, 1])

    out = _epilogue(go0, go1, o_sh, rw.reshape(NTOK, TOPK), aux,
                    w_norm.reshape(1, HDV), Wo)
    return out.reshape(B, S, D)
